# baseline (speedup 1.0000x reference)
import functools

import jax
import jax.numpy as jnp
from jax import lax
from jax.experimental import pallas as pl
from jax.experimental.pallas import tpu as pltpu
from jax.experimental.pallas import tpu_sc as plsc

TOKENS = 8192
D_IN = 2048
D_HID = 1024
N_EXPERTS = 16
TOP_K = 2

BT = 1024

_NC = 2
_NS = 16
_NW = _NC * _NS
_TPW = TOKENS // _NW
_L = 16


def _mlp_body(x_ref, w1_ref, b1_ref, w2_ref, b2_ref, w3_ref, b3_ref,
              logits_ref):
    h = jnp.dot(x_ref[...], w1_ref[...], preferred_element_type=jnp.float32)
    h = jnp.maximum(h + b1_ref[...], 0.0)
    h = jnp.dot(h, w2_ref[...], preferred_element_type=jnp.float32)
    h = jnp.maximum(h + b2_ref[...], 0.0)
    logits = jnp.dot(h, w3_ref[...], preferred_element_type=jnp.float32)
    logits_ref[...] = logits + b3_ref[...]


def _mlp_logits(x, W1, b1, W2, b2, W3, b3):
    grid = (TOKENS // BT,)
    return pl.pallas_call(
        _mlp_body,
        grid=grid,
        in_specs=[
            pl.BlockSpec((BT, D_IN), lambda i: (i, 0)),
            pl.BlockSpec((D_IN, D_HID), lambda i: (0, 0)),
            pl.BlockSpec((1, D_HID), lambda i: (0, 0)),
            pl.BlockSpec((D_HID, D_HID // 2), lambda i: (0, 0)),
            pl.BlockSpec((1, D_HID // 2), lambda i: (0, 0)),
            pl.BlockSpec((D_HID // 2, N_EXPERTS), lambda i: (0, 0)),
            pl.BlockSpec((1, N_EXPERTS), lambda i: (0, 0)),
        ],
        out_specs=pl.BlockSpec((BT, N_EXPERTS), lambda i: (i, 0)),
        out_shape=jax.ShapeDtypeStruct((TOKENS, N_EXPERTS), jnp.float32),
    )(x, W1, b1.reshape(1, -1), W2, b2.reshape(1, -1), W3, b3.reshape(1, -1))


@functools.partial(
    pl.kernel,
    mesh=plsc.VectorSubcoreMesh(core_axis_name="c", subcore_axis_name="s"),
    out_type=[
        jax.ShapeDtypeStruct((TOKENS, N_EXPERTS), jnp.float32),
        jax.ShapeDtypeStruct((TOKENS, TOP_K), jnp.int32),
    ],
    scratch_types=[
        pltpu.VMEM((_TPW, N_EXPERTS), jnp.float32),
        pltpu.VMEM((_TPW, N_EXPERTS), jnp.float32),
        pltpu.VMEM((_TPW, TOP_K), jnp.int32),
    ],
    compiler_params=pltpu.CompilerParams(needs_layout_passes=False),
)
def _gating_sc(logits_hbm, gates_hbm, idx_hbm, lg_v, gt_v, ix_v):
    wid = lax.axis_index("s") * _NC + lax.axis_index("c")
    base = wid * _TPW
    pltpu.sync_copy(logits_hbm.at[pl.ds(base, _TPW)], lg_v)

    neg_inf = jnp.full((_L,), -jnp.inf, jnp.float32)
    zeros_i = jnp.zeros((_L,), jnp.int32)
    lanes = lax.iota(jnp.int32, _L)

    def group(g):
        tok = g * _L + lanes
        m1, i1, m2, i2 = neg_inf, zeros_i, neg_inf, zeros_i
        for e in range(N_EXPERTS):
            e_vec = jnp.full((_L,), e, jnp.int32)
            v = plsc.load_gather(lg_v, [tok, e_vec])
            gt1 = v > m1
            gt2 = jnp.logical_and(v > m2, jnp.logical_not(gt1))
            i2 = jnp.where(gt1, i1, jnp.where(gt2, e_vec, i2))
            m2 = jnp.where(gt1, m1, jnp.where(gt2, v, m2))
            i1 = jnp.where(gt1, e_vec, i1)
            m1 = jnp.where(gt1, v, m1)
        e2 = jnp.exp(m2 - m1)
        denom = 1.0 + e2
        g1 = 1.0 / denom
        g2 = e2 / denom
        for e in range(N_EXPERTS):
            e_vec = jnp.full((_L,), e, jnp.int32)
            col = jnp.where(i1 == e_vec, g1,
                            jnp.where(i2 == e_vec, g2, 0.0))
            plsc.store_scatter(gt_v, [tok, e_vec], col)
        plsc.store_scatter(ix_v, [tok, zeros_i], i1)
        plsc.store_scatter(ix_v, [tok, zeros_i + 1], i2)

    for g in range(_TPW // _L):
        group(g)

    pltpu.sync_copy(gt_v, gates_hbm.at[pl.ds(base, _TPW)])
    pltpu.sync_copy(ix_v, idx_hbm.at[pl.ds(base, _TPW)])


@jax.jit
def kernel(x, W1, b1, W2, b2, W3, b3):
    logits = _mlp_logits(x, W1, b1, W2, b2, W3, b3)
    gates, idx = _gating_sc(logits)
    return gates, idx


# speedup vs baseline: 1.0269x; 1.0269x over previous
import functools

import jax
import jax.numpy as jnp
from jax import lax
from jax.experimental import pallas as pl
from jax.experimental.pallas import tpu as pltpu
from jax.experimental.pallas import tpu_sc as plsc

TOKENS = 8192
D_IN = 2048
D_HID = 1024
N_EXPERTS = 16
TOP_K = 2

BT = 1024

_NC = 2
_NS = 16
_NW = _NC * _NS
_TPW = TOKENS // _NW
_L = 16


def _mlp_body(x_ref, w1_ref, b1_ref, w2_ref, b2_ref, w3_ref, b3_ref,
              logits_ref):
    h = jnp.dot(x_ref[...], w1_ref[...], preferred_element_type=jnp.float32)
    h = jnp.maximum(h + b1_ref[...], 0.0)
    h = jnp.dot(h, w2_ref[...], preferred_element_type=jnp.float32)
    h = jnp.maximum(h + b2_ref[...], 0.0)
    logits = jnp.dot(h, w3_ref[...], preferred_element_type=jnp.float32)
    logits_ref[...] = logits + b3_ref[...]


def _mlp_logits(x, W1, b1, W2, b2, W3, b3):
    grid = (TOKENS // BT,)
    return pl.pallas_call(
        _mlp_body,
        grid=grid,
        in_specs=[
            pl.BlockSpec((BT, D_IN), lambda i: (i, 0)),
            pl.BlockSpec((D_IN, D_HID), lambda i: (0, 0)),
            pl.BlockSpec((1, D_HID), lambda i: (0, 0)),
            pl.BlockSpec((D_HID, D_HID // 2), lambda i: (0, 0)),
            pl.BlockSpec((1, D_HID // 2), lambda i: (0, 0)),
            pl.BlockSpec((D_HID // 2, N_EXPERTS), lambda i: (0, 0)),
            pl.BlockSpec((1, N_EXPERTS), lambda i: (0, 0)),
        ],
        out_specs=pl.BlockSpec((BT, N_EXPERTS), lambda i: (i, 0)),
        out_shape=jax.ShapeDtypeStruct((TOKENS, N_EXPERTS), jnp.float32),
    )(x, W1, b1.reshape(1, -1), W2, b2.reshape(1, -1), W3, b3.reshape(1, -1))


@functools.partial(
    pl.kernel,
    mesh=plsc.VectorSubcoreMesh(core_axis_name="c", subcore_axis_name="s"),
    out_type=[
        jax.ShapeDtypeStruct((TOKENS, N_EXPERTS), jnp.float32),
        jax.ShapeDtypeStruct((TOKENS, TOP_K), jnp.int32),
    ],
    scratch_types=[
        pltpu.VMEM((_TPW, N_EXPERTS), jnp.float32),
        pltpu.VMEM((_TPW, N_EXPERTS), jnp.float32),
        pltpu.VMEM((_TPW, TOP_K), jnp.int32),
    ],
    compiler_params=pltpu.CompilerParams(needs_layout_passes=False),
)
def _gating_sc(logits_hbm, gates_hbm, idx_hbm, lg_v, gt_v, ix_v):
    wid = lax.axis_index("s") * _NC + lax.axis_index("c")
    base = wid * _TPW
    pltpu.sync_copy(logits_hbm.at[pl.ds(base, _TPW)], lg_v)

    neg_inf = jnp.full((_L,), -jnp.inf, jnp.float32)
    zeros_i = jnp.zeros((_L,), jnp.int32)
    lanes = lax.iota(jnp.int32, _L)

    def group(g):
        tok = g * _L + lanes
        m1, i1, m2, i2 = neg_inf, zeros_i, neg_inf, zeros_i
        for e in range(N_EXPERTS):
            e_vec = jnp.full((_L,), e, jnp.int32)
            v = plsc.load_gather(lg_v, [tok, e_vec])
            gt1 = v > m1
            gt2 = jnp.logical_and(v > m2, jnp.logical_not(gt1))
            i2 = jnp.where(gt1, i1, jnp.where(gt2, e_vec, i2))
            m2 = jnp.where(gt1, m1, jnp.where(gt2, v, m2))
            i1 = jnp.where(gt1, e_vec, i1)
            m1 = jnp.where(gt1, v, m1)
        e2 = jnp.exp(m2 - m1)
        denom = 1.0 + e2
        g1 = 1.0 / denom
        g2 = e2 / denom
        for e in range(N_EXPERTS):
            e_vec = jnp.full((_L,), e, jnp.int32)
            col = jnp.where(i1 == e_vec, g1,
                            jnp.where(i2 == e_vec, g2, 0.0))
            plsc.store_scatter(gt_v, [tok, e_vec], col)
        plsc.store_scatter(ix_v, [tok, zeros_i], i1)
        plsc.store_scatter(ix_v, [tok, zeros_i + 1], i2)

    def body(i, carry):
        for j in range(4):
            group(i * 4 + j)
        return carry

    lax.fori_loop(0, _TPW // _L // 4, body, None)

    pltpu.sync_copy(gt_v, gates_hbm.at[pl.ds(base, _TPW)])
    pltpu.sync_copy(ix_v, idx_hbm.at[pl.ds(base, _TPW)])


@jax.jit
def kernel(x, W1, b1, W2, b2, W3, b3):
    logits = _mlp_logits(x, W1, b1, W2, b2, W3, b3)
    gates, idx = _gating_sc(logits)
    return gates, idx


# speedup vs baseline: 1.0310x; 1.0040x over previous
import functools

import jax
import jax.numpy as jnp
from jax import lax
from jax.experimental import pallas as pl
from jax.experimental.pallas import tpu as pltpu
from jax.experimental.pallas import tpu_sc as plsc

TOKENS = 8192
D_IN = 2048
D_HID = 1024
N_EXPERTS = 16
TOP_K = 2

BT = 1024

_NC = 2
_NS = 16
_NW = _NC * _NS
_TPW = TOKENS // _NW
_L = 16


def _mlp_body(x_ref, w1_ref, b1_ref, w2_ref, b2_ref, w3_ref, b3_ref,
              logits_ref):
    h = jnp.dot(x_ref[...], w1_ref[...], preferred_element_type=jnp.float32)
    h = jnp.maximum(h + b1_ref[...], 0.0)
    h = jnp.dot(h, w2_ref[...], preferred_element_type=jnp.float32)
    h = jnp.maximum(h + b2_ref[...], 0.0)
    logits = jnp.dot(h, w3_ref[...], preferred_element_type=jnp.float32)
    logits_ref[...] = logits + b3_ref[...]


def _mlp_logits(x, W1, b1, W2, b2, W3, b3):
    grid = (TOKENS // BT,)
    return pl.pallas_call(
        _mlp_body,
        grid=grid,
        in_specs=[
            pl.BlockSpec((BT, D_IN), lambda i: (i, 0)),
            pl.BlockSpec((D_IN, D_HID), lambda i: (0, 0)),
            pl.BlockSpec((1, D_HID), lambda i: (0, 0)),
            pl.BlockSpec((D_HID, D_HID // 2), lambda i: (0, 0)),
            pl.BlockSpec((1, D_HID // 2), lambda i: (0, 0)),
            pl.BlockSpec((D_HID // 2, N_EXPERTS), lambda i: (0, 0)),
            pl.BlockSpec((1, N_EXPERTS), lambda i: (0, 0)),
        ],
        out_specs=pl.BlockSpec((BT, N_EXPERTS), lambda i: (i, 0)),
        out_shape=jax.ShapeDtypeStruct((TOKENS, N_EXPERTS), jnp.float32),
    )(x, W1, b1.reshape(1, -1), W2, b2.reshape(1, -1), W3, b3.reshape(1, -1))


@functools.partial(
    pl.kernel,
    mesh=plsc.VectorSubcoreMesh(core_axis_name="c", subcore_axis_name="s"),
    out_type=[
        jax.ShapeDtypeStruct((TOKENS, N_EXPERTS), jnp.float32),
        jax.ShapeDtypeStruct((TOKENS, TOP_K), jnp.int32),
    ],
    scratch_types=[
        pltpu.VMEM((_TPW, N_EXPERTS), jnp.float32),
        pltpu.VMEM((_TPW, N_EXPERTS), jnp.float32),
        pltpu.VMEM((_TPW, TOP_K), jnp.int32),
    ],
    compiler_params=pltpu.CompilerParams(needs_layout_passes=False),
)
def _gating_sc(logits_hbm, gates_hbm, idx_hbm, lg_v, gt_v, ix_v):
    wid = lax.axis_index("s") * _NC + lax.axis_index("c")
    base = wid * _TPW
    pltpu.sync_copy(logits_hbm.at[pl.ds(base, _TPW)], lg_v)

    neg_inf = jnp.full((_L,), -jnp.inf, jnp.float32)
    zeros_i = jnp.zeros((_L,), jnp.int32)
    lanes = lax.iota(jnp.int32, _L)

    def group(g, carry):
        tok = g * _L + lanes
        m1, i1, m2, i2 = neg_inf, zeros_i, neg_inf, zeros_i
        for e in range(N_EXPERTS):
            e_vec = jnp.full((_L,), e, jnp.int32)
            v = plsc.load_gather(lg_v, [tok, e_vec])
            gt1 = v > m1
            gt2 = jnp.logical_and(v > m2, jnp.logical_not(gt1))
            i2 = jnp.where(gt1, i1, jnp.where(gt2, e_vec, i2))
            m2 = jnp.where(gt1, m1, jnp.where(gt2, v, m2))
            i1 = jnp.where(gt1, e_vec, i1)
            m1 = jnp.where(gt1, v, m1)
        e2 = jnp.exp(m2 - m1)
        denom = 1.0 + e2
        g1 = 1.0 / denom
        g2 = e2 / denom
        for e in range(N_EXPERTS):
            e_vec = jnp.full((_L,), e, jnp.int32)
            col = jnp.where(i1 == e_vec, g1,
                            jnp.where(i2 == e_vec, g2, 0.0))
            plsc.store_scatter(gt_v, [tok, e_vec], col)
        plsc.store_scatter(ix_v, [tok, zeros_i], i1)
        plsc.store_scatter(ix_v, [tok, zeros_i + 1], i2)
        return carry

    lax.fori_loop(0, _TPW // _L, group, None)

    pltpu.sync_copy(gt_v, gates_hbm.at[pl.ds(base, _TPW)])
    pltpu.sync_copy(ix_v, idx_hbm.at[pl.ds(base, _TPW)])


@jax.jit
def kernel(x, W1, b1, W2, b2, W3, b3):
    logits = _mlp_logits(x, W1, b1, W2, b2, W3, b3)
    gates, idx = _gating_sc(logits)
    return gates, idx
